# baseline (speedup 1.0000x reference)
import functools

import jax
import jax.numpy as jnp
from jax import lax
from jax.experimental import pallas as pl
from jax.experimental.pallas import tpu as pltpu
from jax.experimental.pallas import tpu_sc as plsc

F = 256
FQ = 64
N_PAD = 10240
NPT = 320
E_TOT = 160000
ECH = 2000
NGRP = ECH // 16
NCHUNK = E_TOT // ECH
CAP = 16384
GK = 128
FMAX = 3.4e38


def _mm_body(a_ref, w_ref, b_ref, o_ref):
    o_ref[...] = (
        jnp.dot(a_ref[...], w_ref[...], preferred_element_type=jnp.float32)
        + b_ref[...]
    )


def _mm(a, w, b, bm=512):
    m, k = a.shape
    _, n = w.shape
    return pl.pallas_call(
        _mm_body,
        grid=(m // bm,),
        in_specs=[
            pl.BlockSpec((bm, k), lambda i: (i, 0)),
            pl.BlockSpec((k, n), lambda i: (0, 0)),
            pl.BlockSpec((1, n), lambda i: (0, 0)),
        ],
        out_specs=pl.BlockSpec((bm, n), lambda i: (i, 0)),
        out_shape=jax.ShapeDtypeStruct((m, n), jnp.float32),
    )(a, w, b.reshape(1, -1))


def _sc_body(src_hbm, dst_hbm, bq_hbm,
             out_sum, out_sq, out_mx, out_mn, out_cnt,
             src_v, dst_v, list_v, acc_s, acc_q, acc_mx, acc_mn,
             cnt_acc, idx_b0, dl_b0, rows_v0, idx_b1, dl_b1, rows_v1,
             sem0, sem1):
    nc_ax = lax.axis_index("c")
    ns_ax = lax.axis_index("s")
    wid = ns_ax * 2 + nc_ax
    base = wid * NPT
    iota = lax.iota(jnp.int32, 16)
    z16 = jnp.zeros((16,), jnp.float32)
    ones16 = jnp.ones((16,), jnp.float32)

    def zcnt(i, carry):
        cnt_acc[pl.ds(i * 16, 16)] = z16
        return carry

    lax.fori_loop(0, NPT // 16, zcnt, 0)

    def chunk(ci, off):
        pltpu.sync_copy(src_hbm.at[pl.ds(ci * ECH, ECH)], src_v)
        pltpu.sync_copy(dst_hbm.at[pl.ds(ci * ECH, ECH)], dst_v)

        def grp(g, off):
            s = src_v[pl.ds(g * 16, 16)]
            d = dst_v[pl.ds(g * 16, 16)]
            dl = d - base
            m = (dl >= 0) & (dl < NPT)
            dlc = jnp.where(m, dl, 0)
            plsc.addupdate_scatter(cnt_acc, [dlc], ones16, mask=m)
            packed = (s << 9) | dlc
            offc = jnp.minimum(off, CAP - 16)
            plsc.store_compressed(list_v.at[pl.ds(offc, 16)], packed, mask=m)
            pc = jnp.max(plsc.all_reduce_population_count(m))
            return off + pc

        return lax.fori_loop(0, NGRP, grp, off)

    off = lax.fori_loop(0, NCHUNK, chunk, jnp.int32(0))
    pltpu.sync_copy(cnt_acc, out_cnt.at[pl.ds(base, NPT)])

    offc = jnp.minimum(off, CAP)
    dummy = jnp.full((16,), NPT, jnp.int32)
    for g in range(2 * GK // 16):
        list_v[pl.ds(offc + g * 16, 16)] = dummy
    npair = (offc + 2 * GK - 1) // (2 * GK)

    for q in range(4):
        def zacc(i, carry):
            for k in range(FQ // 16):
                ks = pl.ds(k * 16, 16)
                acc_s[i, ks] = z16
                acc_q[i, ks] = z16
                acc_mx[i, ks] = jnp.full((16,), -FMAX, jnp.float32)
                acc_mn[i, ks] = jnp.full((16,), FMAX, jnp.float32)
            return carry

        lax.fori_loop(0, NPT + 1, zacc, 0)

        def build(ci, idx_ref, dl_ref):
            eb = ci * GK
            for g in range(GK // 16):
                p = list_v[pl.ds(eb + g * 16, 16)]
                idx_ref[pl.ds(g * 16, 16)] = p >> 9
                dl_ref[pl.ds(g * 16, 16)] = p & 511

        def start(ci, idx_ref, rows_ref, sem_):
            pltpu.async_copy(bq_hbm.at[q].at[idx_ref], rows_ref, sem_)

        def wait(idx_ref, rows_ref, sem_):
            pltpu.make_async_copy(
                bq_hbm.at[q].at[idx_ref], rows_ref, sem_).wait()

        def process(rows_ref, dl_ref):
            def one_edge(e):
                dlv = plsc.load_gather(dl_ref, [jnp.full((16,), e, jnp.int32)])
                for k in range(FQ // 16):
                    ks = pl.ds(k * 16, 16)
                    col = iota + (k * 16)
                    r = rows_ref[e, ks]
                    s_ = plsc.load_gather(acc_s, [dlv, col])
                    plsc.store_scatter(acc_s, [dlv, col], s_ + r)
                    q_ = plsc.load_gather(acc_q, [dlv, col])
                    plsc.store_scatter(acc_q, [dlv, col], q_ + r * r)
                    m_ = plsc.load_gather(acc_mx, [dlv, col])
                    plsc.store_scatter(acc_mx, [dlv, col], jnp.maximum(m_, r))
                    n_ = plsc.load_gather(acc_mn, [dlv, col])
                    plsc.store_scatter(acc_mn, [dlv, col], jnp.minimum(n_, r))

            def edge(e2, carry):
                one_edge(e2 * 2)
                one_edge(e2 * 2 + 1)
                return carry

            lax.fori_loop(0, GK // 2, edge, 0)

        build(0, idx_b0, dl_b0)
        start(0, idx_b0, rows_v0, sem0)

        def pair(pi, carry):
            ci0 = pi * 2
            build(ci0 + 1, idx_b1, dl_b1)
            start(ci0 + 1, idx_b1, rows_v1, sem1)
            wait(idx_b0, rows_v0, sem0)
            process(rows_v0, dl_b0)

            @pl.when(pi + 1 < npair)
            def _():
                build(ci0 + 2, idx_b0, dl_b0)
                start(ci0 + 2, idx_b0, rows_v0, sem0)

            wait(idx_b1, rows_v1, sem1)
            process(rows_v1, dl_b1)
            return carry

        lax.fori_loop(0, npair, pair, 0)

        rs = pl.ds(base, NPT)
        pltpu.sync_copy(acc_s.at[0:NPT], out_sum.at[q].at[rs])
        pltpu.sync_copy(acc_q.at[0:NPT], out_sq.at[q].at[rs])
        pltpu.sync_copy(acc_mx.at[0:NPT], out_mx.at[q].at[rs])
        pltpu.sync_copy(acc_mn.at[0:NPT], out_mn.at[q].at[rs])


def _sc_scatter(src, dst, bq):
    f32 = jnp.float32
    agg_t = jax.ShapeDtypeStruct((4, N_PAD, FQ), f32)
    return pl.kernel(
        _sc_body,
        out_type=(agg_t, agg_t, agg_t, agg_t,
                  jax.ShapeDtypeStruct((N_PAD,), f32)),
        mesh=plsc.VectorSubcoreMesh(core_axis_name="c", subcore_axis_name="s"),
        compiler_params=pltpu.CompilerParams(
            needs_layout_passes=False, use_tc_tiling_on_sc=False),
        scratch_types=[
            pltpu.VMEM((ECH,), jnp.int32),
            pltpu.VMEM((ECH,), jnp.int32),
            pltpu.VMEM((CAP + 2 * GK,), jnp.int32),
            pltpu.VMEM((NPT + 1, FQ), f32),
            pltpu.VMEM((NPT + 1, FQ), f32),
            pltpu.VMEM((NPT + 1, FQ), f32),
            pltpu.VMEM((NPT + 1, FQ), f32),
            pltpu.VMEM((NPT,), f32),
            pltpu.VMEM((GK,), jnp.int32),
            pltpu.VMEM((GK,), jnp.int32),
            pltpu.VMEM((GK, FQ), f32),
            pltpu.VMEM((GK,), jnp.int32),
            pltpu.VMEM((GK,), jnp.int32),
            pltpu.VMEM((GK, FQ), f32),
            pltpu.SemaphoreType.DMA,
            pltpu.SemaphoreType.DMA,
        ],
    )(src, dst, bq)


def _comb_body(x_ref, a_ref, cs_ref, cq_ref, cx_ref, cn_ref, cnt_ref,
               w0_ref, wa_ref, wb_ref, wc_ref, bp_ref, wl_ref, bl_ref, o_ref):
    cnt_raw = cnt_ref[...]
    he = cnt_raw > 0.0
    cnt = jnp.maximum(cnt_raw, 1.0)
    inv = 1.0 / cnt
    a = a_ref[...]
    gmean = cs_ref[...] * inv
    mean = jnp.where(he, a + gmean, 0.0)
    var = cq_ref[...] * inv - gmean * gmean
    std = jnp.sqrt(jnp.maximum(var, 0.0) + 1e-5)
    mx = jnp.where(he, a + cx_ref[...], 0.0)
    mn = jnp.where(he, a + cn_ref[...], 0.0)
    agg = jnp.concatenate([mx, mn, mean, std], axis=-1)
    amp = jnp.log(cnt + 1.0)
    att = 1.0 / amp
    dot = functools.partial(jnp.dot, preferred_element_type=jnp.float32)
    h = (dot(x_ref[...], w0_ref[...])
         + dot(agg, wa_ref[...])
         + amp * dot(agg, wb_ref[...])
         + att * dot(agg, wc_ref[...])
         + bp_ref[...])
    o_ref[...] = dot(h, wl_ref[...]) + bl_ref[...]


def _combine(x_pad, a_tab, csum, csq, cmax, cmin, cnt,
             W_post, b_post, W_lin, b_lin, bm=512):
    m = x_pad.shape[0]
    n = W_lin.shape[1]
    blk = lambda r, c: pl.BlockSpec((r, c), lambda i: (i, 0))
    wblk = lambda r, c: pl.BlockSpec((r, c), lambda i: (0, 0))
    return pl.pallas_call(
        _comb_body,
        grid=(m // bm,),
        in_specs=[
            blk(bm, F), blk(bm, F), blk(bm, F), blk(bm, F), blk(bm, F),
            blk(bm, F), blk(bm, 1),
            wblk(F, n), wblk(4 * F, n), wblk(4 * F, n), wblk(4 * F, n),
            wblk(1, n), wblk(F, n), wblk(1, n),
        ],
        out_specs=blk(bm, n),
        out_shape=jax.ShapeDtypeStruct((m, n), jnp.float32),
    )(x_pad, a_tab, csum, csq, cmax, cmin, cnt,
      W_post[:F], W_post[F:5 * F], W_post[5 * F:9 * F], W_post[9 * F:],
      b_post.reshape(1, -1), W_lin, b_lin.reshape(1, -1))


def kernel(x, edge_index, W_pre, b_pre, W_post, b_post, W_lin, b_lin):
    n_nodes, f = x.shape
    src = edge_index[0].astype(jnp.int32)
    dst = edge_index[1].astype(jnp.int32)

    x_pad = jnp.pad(x, ((0, N_PAD - n_nodes), (0, 0)))
    w_cat = jnp.concatenate([W_pre[:f], W_pre[f:]], axis=1)
    b_cat = jnp.concatenate([b_pre, jnp.zeros_like(b_pre)])
    ab = _mm(x_pad, w_cat, b_cat)
    a_tab = ab[:, :f]
    b_tab = ab[:, f:]

    bq = b_tab.reshape(N_PAD, 4, FQ).transpose(1, 0, 2)

    csum4, csq4, cmax4, cmin4, cnt = _sc_scatter(src, dst, bq)
    unq = lambda t: t.transpose(1, 0, 2).reshape(N_PAD, F)
    csum, csq, cmax, cmin = unq(csum4), unq(csq4), unq(cmax4), unq(cmin4)

    out = _combine(x_pad, a_tab, csum, csq, cmax, cmin,
                   cnt.reshape(N_PAD, 1), W_post, b_post, W_lin, b_lin)
    return out[:n_nodes]


# speedup vs baseline: 1.5785x; 1.5785x over previous
import functools

import jax
import jax.numpy as jnp
from jax import lax
from jax.experimental import pallas as pl
from jax.experimental.pallas import tpu as pltpu
from jax.experimental.pallas import tpu_sc as plsc

F = 256
FQ = 64
N_PAD = 10240
NPT = 320
E_TOT = 160000
ECH = 2000
NGRP = ECH // 16
NCHUNK = E_TOT // ECH
CAP = 16384
GK = 128
FMAX = 3.4e38


def _mm_body(a_ref, w_ref, b_ref, o_ref):
    o_ref[...] = (
        jnp.dot(a_ref[...], w_ref[...], preferred_element_type=jnp.float32)
        + b_ref[...]
    )


def _mm(a, w, b, bm=512):
    m, k = a.shape
    _, n = w.shape
    return pl.pallas_call(
        _mm_body,
        grid=(m // bm,),
        in_specs=[
            pl.BlockSpec((bm, k), lambda i: (i, 0)),
            pl.BlockSpec((k, n), lambda i: (0, 0)),
            pl.BlockSpec((1, n), lambda i: (0, 0)),
        ],
        out_specs=pl.BlockSpec((bm, n), lambda i: (i, 0)),
        out_shape=jax.ShapeDtypeStruct((m, n), jnp.float32),
    )(a, w, b.reshape(1, -1))


def _sc_body(src_hbm, dst_hbm, bq_hbm,
             out_sum, out_sq, out_mx, out_mn, out_cnt,
             src_v, dst_v, list_v, acc_s, acc_q, acc_mx, acc_mn,
             cnt_acc, idx_b0, dl_b0, rows_v0, idx_b1, dl_b1, rows_v1,
             sem0, sem1):
    nc_ax = lax.axis_index("c")
    ns_ax = lax.axis_index("s")
    wid = ns_ax * 2 + nc_ax
    base = wid * NPT
    iota = lax.iota(jnp.int32, 16)
    z16 = jnp.zeros((16,), jnp.float32)
    ones16 = jnp.ones((16,), jnp.float32)

    def zcnt(i, carry):
        cnt_acc[pl.ds(i * 16, 16)] = z16
        return carry

    lax.fori_loop(0, NPT // 16, zcnt, 0)

    def chunk(ci, off):
        pltpu.sync_copy(src_hbm.at[pl.ds(ci * ECH, ECH)], src_v)
        pltpu.sync_copy(dst_hbm.at[pl.ds(ci * ECH, ECH)], dst_v)

        def grp(g, off):
            s = src_v[pl.ds(g * 16, 16)]
            d = dst_v[pl.ds(g * 16, 16)]
            dl = d - base
            m = (dl >= 0) & (dl < NPT)
            dlc = jnp.where(m, dl, 0)
            plsc.addupdate_scatter(cnt_acc, [dlc], ones16, mask=m)
            packed = (s << 9) | dlc
            offc = jnp.minimum(off, CAP - 16)
            plsc.store_compressed(list_v.at[pl.ds(offc, 16)], packed, mask=m)
            pc = jnp.max(plsc.all_reduce_population_count(m))
            return off + pc

        return lax.fori_loop(0, NGRP, grp, off)

    off = lax.fori_loop(0, NCHUNK, chunk, jnp.int32(0))
    pltpu.sync_copy(cnt_acc, out_cnt.at[pl.ds(base, NPT)])

    offc = jnp.minimum(off, CAP)
    dummy = jnp.full((16,), NPT, jnp.int32)
    for g in range(2 * GK // 16):
        list_v[pl.ds(offc + g * 16, 16)] = dummy
    npair = (offc + 2 * GK - 1) // (2 * GK)

    for q in range(4):
        def zacc(i, carry):
            for k in range(FQ // 16):
                ks = pl.ds(k * 16, 16)
                acc_s[i, ks] = z16
                acc_q[i, ks] = z16
                acc_mx[i, ks] = jnp.full((16,), -FMAX, jnp.float32)
                acc_mn[i, ks] = jnp.full((16,), FMAX, jnp.float32)
            return carry

        lax.fori_loop(0, NPT + 1, zacc, 0)

        def build(ci, idx_ref, dl_ref):
            eb = ci * GK
            for g in range(GK // 16):
                p = list_v[pl.ds(eb + g * 16, 16)]
                idx_ref[pl.ds(g * 16, 16)] = p >> 9
                dl_ref[pl.ds(g * 16, 16)] = p & 511

        def start(ci, idx_ref, rows_ref, sem_):
            pltpu.async_copy(bq_hbm.at[q].at[idx_ref], rows_ref, sem_)

        def wait(idx_ref, rows_ref, sem_):
            pltpu.make_async_copy(
                bq_hbm.at[q].at[idx_ref], rows_ref, sem_).wait()

        def process(rows_ref, dl_ref):
            def one_edge(e):
                dlg = dl_ref[pl.ds(e & -16, 16)]
                dl = jnp.max(jnp.where(iota == (e & 15), dlg, 0))
                for k in range(FQ // 16):
                    ks = pl.ds(k * 16, 16)
                    r = rows_ref[e, ks]
                    acc_s[dl, ks] = acc_s[dl, ks] + r
                    acc_q[dl, ks] = acc_q[dl, ks] + r * r
                    acc_mx[dl, ks] = jnp.maximum(acc_mx[dl, ks], r)
                    acc_mn[dl, ks] = jnp.minimum(acc_mn[dl, ks], r)

            def edge(e2, carry):
                one_edge(e2 * 2)
                one_edge(e2 * 2 + 1)
                return carry

            lax.fori_loop(0, GK // 2, edge, 0)

        build(0, idx_b0, dl_b0)
        start(0, idx_b0, rows_v0, sem0)

        def pair(pi, carry):
            ci0 = pi * 2
            build(ci0 + 1, idx_b1, dl_b1)
            start(ci0 + 1, idx_b1, rows_v1, sem1)
            wait(idx_b0, rows_v0, sem0)
            process(rows_v0, dl_b0)

            @pl.when(pi + 1 < npair)
            def _():
                build(ci0 + 2, idx_b0, dl_b0)
                start(ci0 + 2, idx_b0, rows_v0, sem0)

            wait(idx_b1, rows_v1, sem1)
            process(rows_v1, dl_b1)
            return carry

        lax.fori_loop(0, npair, pair, 0)

        rs = pl.ds(base, NPT)
        pltpu.sync_copy(acc_s.at[0:NPT], out_sum.at[q].at[rs])
        pltpu.sync_copy(acc_q.at[0:NPT], out_sq.at[q].at[rs])
        pltpu.sync_copy(acc_mx.at[0:NPT], out_mx.at[q].at[rs])
        pltpu.sync_copy(acc_mn.at[0:NPT], out_mn.at[q].at[rs])


def _sc_scatter(src, dst, bq):
    f32 = jnp.float32
    agg_t = jax.ShapeDtypeStruct((4, N_PAD, FQ), f32)
    return pl.kernel(
        _sc_body,
        out_type=(agg_t, agg_t, agg_t, agg_t,
                  jax.ShapeDtypeStruct((N_PAD,), f32)),
        mesh=plsc.VectorSubcoreMesh(core_axis_name="c", subcore_axis_name="s"),
        compiler_params=pltpu.CompilerParams(
            needs_layout_passes=False, use_tc_tiling_on_sc=False),
        scratch_types=[
            pltpu.VMEM((ECH,), jnp.int32),
            pltpu.VMEM((ECH,), jnp.int32),
            pltpu.VMEM((CAP + 2 * GK,), jnp.int32),
            pltpu.VMEM((NPT + 1, FQ), f32),
            pltpu.VMEM((NPT + 1, FQ), f32),
            pltpu.VMEM((NPT + 1, FQ), f32),
            pltpu.VMEM((NPT + 1, FQ), f32),
            pltpu.VMEM((NPT,), f32),
            pltpu.VMEM((GK,), jnp.int32),
            pltpu.VMEM((GK,), jnp.int32),
            pltpu.VMEM((GK, FQ), f32),
            pltpu.VMEM((GK,), jnp.int32),
            pltpu.VMEM((GK,), jnp.int32),
            pltpu.VMEM((GK, FQ), f32),
            pltpu.SemaphoreType.DMA,
            pltpu.SemaphoreType.DMA,
        ],
    )(src, dst, bq)


def _comb_body(x_ref, a_ref, cs_ref, cq_ref, cx_ref, cn_ref, cnt_ref,
               w0_ref, wa_ref, wb_ref, wc_ref, bp_ref, wl_ref, bl_ref, o_ref):
    cnt_raw = cnt_ref[...]
    he = cnt_raw > 0.0
    cnt = jnp.maximum(cnt_raw, 1.0)
    inv = 1.0 / cnt
    a = a_ref[...]
    gmean = cs_ref[...] * inv
    mean = jnp.where(he, a + gmean, 0.0)
    var = cq_ref[...] * inv - gmean * gmean
    std = jnp.sqrt(jnp.maximum(var, 0.0) + 1e-5)
    mx = jnp.where(he, a + cx_ref[...], 0.0)
    mn = jnp.where(he, a + cn_ref[...], 0.0)
    agg = jnp.concatenate([mx, mn, mean, std], axis=-1)
    amp = jnp.log(cnt + 1.0)
    att = 1.0 / amp
    dot = functools.partial(jnp.dot, preferred_element_type=jnp.float32)
    h = (dot(x_ref[...], w0_ref[...])
         + dot(agg, wa_ref[...])
         + amp * dot(agg, wb_ref[...])
         + att * dot(agg, wc_ref[...])
         + bp_ref[...])
    o_ref[...] = dot(h, wl_ref[...]) + bl_ref[...]


def _combine(x_pad, a_tab, csum, csq, cmax, cmin, cnt,
             W_post, b_post, W_lin, b_lin, bm=512):
    m = x_pad.shape[0]
    n = W_lin.shape[1]
    blk = lambda r, c: pl.BlockSpec((r, c), lambda i: (i, 0))
    wblk = lambda r, c: pl.BlockSpec((r, c), lambda i: (0, 0))
    return pl.pallas_call(
        _comb_body,
        grid=(m // bm,),
        in_specs=[
            blk(bm, F), blk(bm, F), blk(bm, F), blk(bm, F), blk(bm, F),
            blk(bm, F), blk(bm, 1),
            wblk(F, n), wblk(4 * F, n), wblk(4 * F, n), wblk(4 * F, n),
            wblk(1, n), wblk(F, n), wblk(1, n),
        ],
        out_specs=blk(bm, n),
        out_shape=jax.ShapeDtypeStruct((m, n), jnp.float32),
    )(x_pad, a_tab, csum, csq, cmax, cmin, cnt,
      W_post[:F], W_post[F:5 * F], W_post[5 * F:9 * F], W_post[9 * F:],
      b_post.reshape(1, -1), W_lin, b_lin.reshape(1, -1))


def kernel(x, edge_index, W_pre, b_pre, W_post, b_post, W_lin, b_lin):
    n_nodes, f = x.shape
    src = edge_index[0].astype(jnp.int32)
    dst = edge_index[1].astype(jnp.int32)

    x_pad = jnp.pad(x, ((0, N_PAD - n_nodes), (0, 0)))
    w_cat = jnp.concatenate([W_pre[:f], W_pre[f:]], axis=1)
    b_cat = jnp.concatenate([b_pre, jnp.zeros_like(b_pre)])
    ab = _mm(x_pad, w_cat, b_cat)
    a_tab = ab[:, :f]
    b_tab = ab[:, f:]

    bq = b_tab.reshape(N_PAD, 4, FQ).transpose(1, 0, 2)

    csum4, csq4, cmax4, cmin4, cnt = _sc_scatter(src, dst, bq)
    unq = lambda t: t.transpose(1, 0, 2).reshape(N_PAD, F)
    csum, csq, cmax, cmin = unq(csum4), unq(csq4), unq(cmax4), unq(cmin4)

    out = _combine(x_pad, a_tab, csum, csq, cmax, cmin,
                   cnt.reshape(N_PAD, 1), W_post, b_post, W_lin, b_lin)
    return out[:n_nodes]


# speedup vs baseline: 1.7152x; 1.0866x over previous
import functools

import jax
import jax.numpy as jnp
from jax import lax
from jax.experimental import pallas as pl
from jax.experimental.pallas import tpu as pltpu
from jax.experimental.pallas import tpu_sc as plsc

F = 256
FQ = 64
N_PAD = 10240
NPT = 320
E_TOT = 160000
ECH = 2000
NGRP = ECH // 16
NCHUNK = E_TOT // ECH
CAP = 16384
GK = 128
FMAX = 3.4e38


def _mm_body(a_ref, w_ref, b_ref, o_ref):
    o_ref[...] = (
        jnp.dot(a_ref[...], w_ref[...], preferred_element_type=jnp.float32)
        + b_ref[...]
    )


def _mm(a, w, b, bm=512):
    m, k = a.shape
    _, n = w.shape
    return pl.pallas_call(
        _mm_body,
        grid=(m // bm,),
        in_specs=[
            pl.BlockSpec((bm, k), lambda i: (i, 0)),
            pl.BlockSpec((k, n), lambda i: (0, 0)),
            pl.BlockSpec((1, n), lambda i: (0, 0)),
        ],
        out_specs=pl.BlockSpec((bm, n), lambda i: (i, 0)),
        out_shape=jax.ShapeDtypeStruct((m, n), jnp.float32),
    )(a, w, b.reshape(1, -1))


def _sc_body(src_hbm, dst_hbm, bq_hbm,
             out_sum, out_sq, out_mx, out_mn, out_cnt,
             src_v, dst_v, list_v, acc_s, acc_q, acc_mx, acc_mn,
             cnt_acc, idx_b0, dl_b0, rows_v0, idx_b1, dl_b1, rows_v1,
             sem0, sem1):
    nc_ax = lax.axis_index("c")
    ns_ax = lax.axis_index("s")
    wid = ns_ax * 2 + nc_ax
    base = wid * NPT
    iota = lax.iota(jnp.int32, 16)
    z16 = jnp.zeros((16,), jnp.float32)
    ones16 = jnp.ones((16,), jnp.float32)

    def zcnt(i, carry):
        cnt_acc[pl.ds(i * 16, 16)] = z16
        return carry

    lax.fori_loop(0, NPT // 16, zcnt, 0)

    def chunk(ci, off):
        pltpu.sync_copy(src_hbm.at[pl.ds(ci * ECH, ECH)], src_v)
        pltpu.sync_copy(dst_hbm.at[pl.ds(ci * ECH, ECH)], dst_v)

        def grp(g, off):
            s = src_v[pl.ds(g * 16, 16)]
            d = dst_v[pl.ds(g * 16, 16)]
            dl = d - base
            m = (dl >= 0) & (dl < NPT)
            dlc = jnp.where(m, dl, 0)
            plsc.addupdate_scatter(cnt_acc, [dlc], ones16, mask=m)
            packed = (s << 9) | dlc
            offc = jnp.minimum(off, CAP - 16)
            plsc.store_compressed(list_v.at[pl.ds(offc, 16)], packed, mask=m)
            pc = plsc.all_reduce_population_count(m)[0]
            return off + pc

        return lax.fori_loop(0, NGRP, grp, off)

    off = lax.fori_loop(0, NCHUNK, chunk, jnp.int32(0))
    pltpu.sync_copy(cnt_acc, out_cnt.at[pl.ds(base, NPT)])

    offc = jnp.minimum(off, CAP)
    dummy = jnp.full((16,), NPT, jnp.int32)
    for g in range(2 * GK // 16):
        list_v[pl.ds(offc + g * 16, 16)] = dummy
    npair = (offc + 2 * GK - 1) // (2 * GK)

    for q in range(4):
        def zacc(i, carry):
            for k in range(FQ // 16):
                ks = pl.ds(k * 16, 16)
                acc_s[i, ks] = z16
                acc_q[i, ks] = z16
                acc_mx[i, ks] = jnp.full((16,), -FMAX, jnp.float32)
                acc_mn[i, ks] = jnp.full((16,), FMAX, jnp.float32)
            return carry

        lax.fori_loop(0, NPT + 1, zacc, 0)

        def build(ci, idx_ref, dl_ref):
            eb = ci * GK
            for g in range(GK // 16):
                p = list_v[pl.ds(eb + g * 16, 16)]
                idx_ref[pl.ds(g * 16, 16)] = p >> 9
                dl_ref[pl.ds(g * 16, 16)] = p & 511

        def start(ci, idx_ref, rows_ref, sem_):
            pltpu.async_copy(bq_hbm.at[q].at[idx_ref], rows_ref, sem_)

        def wait(idx_ref, rows_ref, sem_):
            pltpu.make_async_copy(
                bq_hbm.at[q].at[idx_ref], rows_ref, sem_).wait()

        def process(rows_ref, dl_ref):
            def one_edge(e):
                dl = dl_ref[pl.ds(e, 16)][0]
                for k in range(FQ // 16):
                    ks = pl.ds(k * 16, 16)
                    r = rows_ref[e, ks]
                    acc_s[dl, ks] = acc_s[dl, ks] + r
                    acc_q[dl, ks] = acc_q[dl, ks] + r * r
                    acc_mx[dl, ks] = jnp.maximum(acc_mx[dl, ks], r)
                    acc_mn[dl, ks] = jnp.minimum(acc_mn[dl, ks], r)

            def edge(e2, carry):
                one_edge(e2 * 2)
                one_edge(e2 * 2 + 1)
                return carry

            lax.fori_loop(0, GK // 2, edge, 0)

        build(0, idx_b0, dl_b0)
        start(0, idx_b0, rows_v0, sem0)

        def pair(pi, carry):
            ci0 = pi * 2
            build(ci0 + 1, idx_b1, dl_b1)
            start(ci0 + 1, idx_b1, rows_v1, sem1)
            wait(idx_b0, rows_v0, sem0)
            process(rows_v0, dl_b0)

            @pl.when(pi + 1 < npair)
            def _():
                build(ci0 + 2, idx_b0, dl_b0)
                start(ci0 + 2, idx_b0, rows_v0, sem0)

            wait(idx_b1, rows_v1, sem1)
            process(rows_v1, dl_b1)
            return carry

        lax.fori_loop(0, npair, pair, 0)

        rs = pl.ds(base, NPT)
        pltpu.sync_copy(acc_s.at[0:NPT], out_sum.at[q].at[rs])
        pltpu.sync_copy(acc_q.at[0:NPT], out_sq.at[q].at[rs])
        pltpu.sync_copy(acc_mx.at[0:NPT], out_mx.at[q].at[rs])
        pltpu.sync_copy(acc_mn.at[0:NPT], out_mn.at[q].at[rs])


def _sc_scatter(src, dst, bq):
    f32 = jnp.float32
    agg_t = jax.ShapeDtypeStruct((4, N_PAD, FQ), f32)
    return pl.kernel(
        _sc_body,
        out_type=(agg_t, agg_t, agg_t, agg_t,
                  jax.ShapeDtypeStruct((N_PAD,), f32)),
        mesh=plsc.VectorSubcoreMesh(core_axis_name="c", subcore_axis_name="s"),
        compiler_params=pltpu.CompilerParams(
            needs_layout_passes=False, use_tc_tiling_on_sc=False),
        scratch_types=[
            pltpu.VMEM((ECH,), jnp.int32),
            pltpu.VMEM((ECH,), jnp.int32),
            pltpu.VMEM((CAP + 2 * GK,), jnp.int32),
            pltpu.VMEM((NPT + 1, FQ), f32),
            pltpu.VMEM((NPT + 1, FQ), f32),
            pltpu.VMEM((NPT + 1, FQ), f32),
            pltpu.VMEM((NPT + 1, FQ), f32),
            pltpu.VMEM((NPT,), f32),
            pltpu.VMEM((GK,), jnp.int32),
            pltpu.VMEM((GK + 16,), jnp.int32),
            pltpu.VMEM((GK, FQ), f32),
            pltpu.VMEM((GK,), jnp.int32),
            pltpu.VMEM((GK + 16,), jnp.int32),
            pltpu.VMEM((GK, FQ), f32),
            pltpu.SemaphoreType.DMA,
            pltpu.SemaphoreType.DMA,
        ],
    )(src, dst, bq)


def _comb_body(x_ref, a_ref, cs_ref, cq_ref, cx_ref, cn_ref, cnt_ref,
               w0_ref, wa_ref, wb_ref, wc_ref, bp_ref, wl_ref, bl_ref, o_ref):
    cnt_raw = cnt_ref[...]
    he = cnt_raw > 0.0
    cnt = jnp.maximum(cnt_raw, 1.0)
    inv = 1.0 / cnt
    a = a_ref[...]
    gmean = cs_ref[...] * inv
    mean = jnp.where(he, a + gmean, 0.0)
    var = cq_ref[...] * inv - gmean * gmean
    std = jnp.sqrt(jnp.maximum(var, 0.0) + 1e-5)
    mx = jnp.where(he, a + cx_ref[...], 0.0)
    mn = jnp.where(he, a + cn_ref[...], 0.0)
    agg = jnp.concatenate([mx, mn, mean, std], axis=-1)
    amp = jnp.log(cnt + 1.0)
    att = 1.0 / amp
    dot = functools.partial(jnp.dot, preferred_element_type=jnp.float32)
    h = (dot(x_ref[...], w0_ref[...])
         + dot(agg, wa_ref[...])
         + amp * dot(agg, wb_ref[...])
         + att * dot(agg, wc_ref[...])
         + bp_ref[...])
    o_ref[...] = dot(h, wl_ref[...]) + bl_ref[...]


def _combine(x_pad, a_tab, csum, csq, cmax, cmin, cnt,
             W_post, b_post, W_lin, b_lin, bm=512):
    m = x_pad.shape[0]
    n = W_lin.shape[1]
    blk = lambda r, c: pl.BlockSpec((r, c), lambda i: (i, 0))
    wblk = lambda r, c: pl.BlockSpec((r, c), lambda i: (0, 0))
    return pl.pallas_call(
        _comb_body,
        grid=(m // bm,),
        in_specs=[
            blk(bm, F), blk(bm, F), blk(bm, F), blk(bm, F), blk(bm, F),
            blk(bm, F), blk(bm, 1),
            wblk(F, n), wblk(4 * F, n), wblk(4 * F, n), wblk(4 * F, n),
            wblk(1, n), wblk(F, n), wblk(1, n),
        ],
        out_specs=blk(bm, n),
        out_shape=jax.ShapeDtypeStruct((m, n), jnp.float32),
    )(x_pad, a_tab, csum, csq, cmax, cmin, cnt,
      W_post[:F], W_post[F:5 * F], W_post[5 * F:9 * F], W_post[9 * F:],
      b_post.reshape(1, -1), W_lin, b_lin.reshape(1, -1))


def kernel(x, edge_index, W_pre, b_pre, W_post, b_post, W_lin, b_lin):
    n_nodes, f = x.shape
    src = edge_index[0].astype(jnp.int32)
    dst = edge_index[1].astype(jnp.int32)

    x_pad = jnp.pad(x, ((0, N_PAD - n_nodes), (0, 0)))
    w_cat = jnp.concatenate([W_pre[:f], W_pre[f:]], axis=1)
    b_cat = jnp.concatenate([b_pre, jnp.zeros_like(b_pre)])
    ab = _mm(x_pad, w_cat, b_cat)
    a_tab = ab[:, :f]
    b_tab = ab[:, f:]

    bq = b_tab.reshape(N_PAD, 4, FQ).transpose(1, 0, 2)

    csum4, csq4, cmax4, cmin4, cnt = _sc_scatter(src, dst, bq)
    unq = lambda t: t.transpose(1, 0, 2).reshape(N_PAD, F)
    csum, csq, cmax, cmin = unq(csum4), unq(csq4), unq(cmax4), unq(cmin4)

    out = _combine(x_pad, a_tab, csum, csq, cmax, cmin,
                   cnt.reshape(N_PAD, 1), W_post, b_post, W_lin, b_lin)
    return out[:n_nodes]


# speedup vs baseline: 2.0733x; 1.2088x over previous
import functools

import jax
import jax.numpy as jnp
from jax import lax
from jax.experimental import pallas as pl
from jax.experimental.pallas import tpu as pltpu
from jax.experimental.pallas import tpu_sc as plsc

F = 256
FQ = 64
N_PAD = 10240
NPT = 320
E_TOT = 160000
ECH = 4000
NGRP = ECH // 16
NCHUNK = E_TOT // ECH
CAP = 16384
GK = 128
FMAX = 3.4e38


def _mm_body(a_ref, w_ref, b_ref, o_ref):
    o_ref[...] = (
        jnp.dot(a_ref[...], w_ref[...], preferred_element_type=jnp.float32)
        + b_ref[...]
    )


def _mm(a, w, b, bm=512):
    m, k = a.shape
    _, n = w.shape
    return pl.pallas_call(
        _mm_body,
        grid=(m // bm,),
        in_specs=[
            pl.BlockSpec((bm, k), lambda i: (i, 0)),
            pl.BlockSpec((k, n), lambda i: (0, 0)),
            pl.BlockSpec((1, n), lambda i: (0, 0)),
        ],
        out_specs=pl.BlockSpec((bm, n), lambda i: (i, 0)),
        out_shape=jax.ShapeDtypeStruct((m, n), jnp.float32),
    )(a, w, b.reshape(1, -1))


def _sc_body(src_hbm, dst_hbm, bq_hbm,
             out_sum, out_sq, out_mx, out_mn, out_cnt,
             src_v, dst_v, list_v, acc_s, acc_q, acc_mx, acc_mn,
             cnt_acc, idx_b0, dl_b0, rows_v0, idx_b1, dl_b1, rows_v1,
             sem0, sem1):
    nc_ax = lax.axis_index("c")
    ns_ax = lax.axis_index("s")
    wid = ns_ax * 2 + nc_ax
    base = wid * NPT
    iota = lax.iota(jnp.int32, 16)
    z16 = jnp.zeros((16,), jnp.float32)
    ones16 = jnp.ones((16,), jnp.float32)

    def zcnt(i, carry):
        cnt_acc[pl.ds(i * 16, 16)] = z16
        return carry

    lax.fori_loop(0, NPT // 16, zcnt, 0)

    def chunk(ci, off):
        pltpu.sync_copy(src_hbm.at[pl.ds(ci * ECH, ECH)], src_v)
        pltpu.sync_copy(dst_hbm.at[pl.ds(ci * ECH, ECH)], dst_v)

        def grp(g, off):
            s = src_v[pl.ds(g * 16, 16)]
            d = dst_v[pl.ds(g * 16, 16)]
            dl = d - base
            m = (dl >= 0) & (dl < NPT)
            dlc = jnp.where(m, dl, 0)
            plsc.addupdate_scatter(cnt_acc, [dlc], ones16, mask=m)
            packed = (s << 9) | dlc
            offc = jnp.minimum(off, CAP - 16)
            plsc.store_compressed(list_v.at[pl.ds(offc, 16)], packed, mask=m)
            pc = plsc.all_reduce_population_count(m)[0]
            return off + pc

        return lax.fori_loop(0, NGRP, grp, off)

    off = lax.fori_loop(0, NCHUNK, chunk, jnp.int32(0))
    pltpu.sync_copy(cnt_acc, out_cnt.at[pl.ds(base, NPT)])

    offc = jnp.minimum(off, CAP)
    dummy = jnp.full((16,), NPT, jnp.int32)
    for g in range(2 * GK // 16):
        list_v[pl.ds(offc + g * 16, 16)] = dummy
    npair = (offc + 2 * GK - 1) // (2 * GK)

    for q in range(4):
        def zacc(i, carry):
            for k in range(FQ // 16):
                ks = pl.ds(k * 16, 16)
                acc_s[i, ks] = z16
                acc_q[i, ks] = z16
                acc_mx[i, ks] = jnp.full((16,), -FMAX, jnp.float32)
                acc_mn[i, ks] = jnp.full((16,), FMAX, jnp.float32)
            return carry

        lax.fori_loop(0, NPT + 1, zacc, 0)

        def build(ci, idx_ref, dl_ref):
            eb = ci * GK
            for g in range(GK // 16):
                p = list_v[pl.ds(eb + g * 16, 16)]
                idx_ref[pl.ds(g * 16, 16)] = p >> 9
                dl_ref[pl.ds(g * 16, 16)] = p & 511

        def start(ci, idx_ref, rows_ref, sem_):
            pltpu.async_copy(bq_hbm.at[q].at[idx_ref], rows_ref, sem_)

        def wait(idx_ref, rows_ref, sem_):
            pltpu.make_async_copy(
                bq_hbm.at[q].at[idx_ref], rows_ref, sem_).wait()

        def process(rows_ref, dl_ref):
            def grp16(g, carry):
                base_e = g * 16
                dlg = dl_ref[pl.ds(base_e, 16)]
                for j in range(16):
                    dl = dlg[j]
                    e = base_e + j
                    for k in range(FQ // 16):
                        ks = pl.ds(k * 16, 16)
                        r = rows_ref[e, ks]
                        acc_s[dl, ks] = acc_s[dl, ks] + r
                        acc_q[dl, ks] = acc_q[dl, ks] + r * r
                        acc_mx[dl, ks] = jnp.maximum(acc_mx[dl, ks], r)
                        acc_mn[dl, ks] = jnp.minimum(acc_mn[dl, ks], r)
                return carry

            lax.fori_loop(0, GK // 16, grp16, 0)

        build(0, idx_b0, dl_b0)
        start(0, idx_b0, rows_v0, sem0)

        def pair(pi, carry):
            ci0 = pi * 2
            build(ci0 + 1, idx_b1, dl_b1)
            start(ci0 + 1, idx_b1, rows_v1, sem1)
            wait(idx_b0, rows_v0, sem0)
            process(rows_v0, dl_b0)

            @pl.when(pi + 1 < npair)
            def _():
                build(ci0 + 2, idx_b0, dl_b0)
                start(ci0 + 2, idx_b0, rows_v0, sem0)

            wait(idx_b1, rows_v1, sem1)
            process(rows_v1, dl_b1)
            return carry

        lax.fori_loop(0, npair, pair, 0)

        rs = pl.ds(base, NPT)
        pltpu.sync_copy(acc_s.at[0:NPT], out_sum.at[q].at[rs])
        pltpu.sync_copy(acc_q.at[0:NPT], out_sq.at[q].at[rs])
        pltpu.sync_copy(acc_mx.at[0:NPT], out_mx.at[q].at[rs])
        pltpu.sync_copy(acc_mn.at[0:NPT], out_mn.at[q].at[rs])


def _sc_scatter(src, dst, bq):
    f32 = jnp.float32
    agg_t = jax.ShapeDtypeStruct((4, N_PAD, FQ), f32)
    return pl.kernel(
        _sc_body,
        out_type=(agg_t, agg_t, agg_t, agg_t,
                  jax.ShapeDtypeStruct((N_PAD,), f32)),
        mesh=plsc.VectorSubcoreMesh(core_axis_name="c", subcore_axis_name="s"),
        compiler_params=pltpu.CompilerParams(
            needs_layout_passes=False, use_tc_tiling_on_sc=False),
        scratch_types=[
            pltpu.VMEM((ECH,), jnp.int32),
            pltpu.VMEM((ECH,), jnp.int32),
            pltpu.VMEM((CAP + 2 * GK,), jnp.int32),
            pltpu.VMEM((NPT + 1, FQ), f32),
            pltpu.VMEM((NPT + 1, FQ), f32),
            pltpu.VMEM((NPT + 1, FQ), f32),
            pltpu.VMEM((NPT + 1, FQ), f32),
            pltpu.VMEM((NPT,), f32),
            pltpu.VMEM((GK,), jnp.int32),
            pltpu.VMEM((GK + 16,), jnp.int32),
            pltpu.VMEM((GK, FQ), f32),
            pltpu.VMEM((GK,), jnp.int32),
            pltpu.VMEM((GK + 16,), jnp.int32),
            pltpu.VMEM((GK, FQ), f32),
            pltpu.SemaphoreType.DMA,
            pltpu.SemaphoreType.DMA,
        ],
    )(src, dst, bq)


def _comb_body(x_ref, a_ref, cs_ref, cq_ref, cx_ref, cn_ref, cnt_ref,
               w0_ref, wa_ref, wb_ref, wc_ref, bp_ref, wl_ref, bl_ref, o_ref):
    cnt_raw = cnt_ref[...]
    he = cnt_raw > 0.0
    cnt = jnp.maximum(cnt_raw, 1.0)
    inv = 1.0 / cnt
    a = a_ref[...]
    gmean = cs_ref[...] * inv
    mean = jnp.where(he, a + gmean, 0.0)
    var = cq_ref[...] * inv - gmean * gmean
    std = jnp.sqrt(jnp.maximum(var, 0.0) + 1e-5)
    mx = jnp.where(he, a + cx_ref[...], 0.0)
    mn = jnp.where(he, a + cn_ref[...], 0.0)
    agg = jnp.concatenate([mx, mn, mean, std], axis=-1)
    amp = jnp.log(cnt + 1.0)
    att = 1.0 / amp
    dot = functools.partial(jnp.dot, preferred_element_type=jnp.float32)
    h = (dot(x_ref[...], w0_ref[...])
         + dot(agg, wa_ref[...])
         + amp * dot(agg, wb_ref[...])
         + att * dot(agg, wc_ref[...])
         + bp_ref[...])
    o_ref[...] = dot(h, wl_ref[...]) + bl_ref[...]


def _combine(x_pad, a_tab, csum, csq, cmax, cmin, cnt,
             W_post, b_post, W_lin, b_lin, bm=512):
    m = x_pad.shape[0]
    n = W_lin.shape[1]
    blk = lambda r, c: pl.BlockSpec((r, c), lambda i: (i, 0))
    wblk = lambda r, c: pl.BlockSpec((r, c), lambda i: (0, 0))
    return pl.pallas_call(
        _comb_body,
        grid=(m // bm,),
        in_specs=[
            blk(bm, F), blk(bm, F), blk(bm, F), blk(bm, F), blk(bm, F),
            blk(bm, F), blk(bm, 1),
            wblk(F, n), wblk(4 * F, n), wblk(4 * F, n), wblk(4 * F, n),
            wblk(1, n), wblk(F, n), wblk(1, n),
        ],
        out_specs=blk(bm, n),
        out_shape=jax.ShapeDtypeStruct((m, n), jnp.float32),
    )(x_pad, a_tab, csum, csq, cmax, cmin, cnt,
      W_post[:F], W_post[F:5 * F], W_post[5 * F:9 * F], W_post[9 * F:],
      b_post.reshape(1, -1), W_lin, b_lin.reshape(1, -1))


def kernel(x, edge_index, W_pre, b_pre, W_post, b_post, W_lin, b_lin):
    n_nodes, f = x.shape
    src = edge_index[0].astype(jnp.int32)
    dst = edge_index[1].astype(jnp.int32)

    x_pad = jnp.pad(x, ((0, N_PAD - n_nodes), (0, 0)))
    w_cat = jnp.concatenate([W_pre[:f], W_pre[f:]], axis=1)
    b_cat = jnp.concatenate([b_pre, jnp.zeros_like(b_pre)])
    ab = _mm(x_pad, w_cat, b_cat)
    a_tab = ab[:, :f]
    b_tab = ab[:, f:]

    bq = b_tab.reshape(N_PAD, 4, FQ).transpose(1, 0, 2)

    csum4, csq4, cmax4, cmin4, cnt = _sc_scatter(src, dst, bq)
    unq = lambda t: t.transpose(1, 0, 2).reshape(N_PAD, F)
    csum, csq, cmax, cmin = unq(csum4), unq(csq4), unq(cmax4), unq(cmin4)

    out = _combine(x_pad, a_tab, csum, csq, cmax, cmin,
                   cnt.reshape(N_PAD, 1), W_post, b_post, W_lin, b_lin)
    return out[:n_nodes]


# speedup vs baseline: 2.1529x; 1.0384x over previous
import functools

import jax
import jax.numpy as jnp
from jax import lax
from jax.experimental import pallas as pl
from jax.experimental.pallas import tpu as pltpu
from jax.experimental.pallas import tpu_sc as plsc

F = 256
FQ = 64
N_PAD = 10240
NPT = 320
E_TOT = 160000
ECH = 4000
NGRP = ECH // 16
NCHUNK = E_TOT // ECH
CAP = 16384
GK = 128
FMAX = 3.4e38


def _mm_body(a_ref, w_ref, b_ref, o_ref):
    o_ref[...] = (
        jnp.dot(a_ref[...], w_ref[...], preferred_element_type=jnp.float32)
        + b_ref[...]
    )


def _mm(a, w, b, bm=512):
    m, k = a.shape
    _, n = w.shape
    return pl.pallas_call(
        _mm_body,
        grid=(m // bm,),
        in_specs=[
            pl.BlockSpec((bm, k), lambda i: (i, 0)),
            pl.BlockSpec((k, n), lambda i: (0, 0)),
            pl.BlockSpec((1, n), lambda i: (0, 0)),
        ],
        out_specs=pl.BlockSpec((bm, n), lambda i: (i, 0)),
        out_shape=jax.ShapeDtypeStruct((m, n), jnp.float32),
    )(a, w, b.reshape(1, -1))


def _sc_body(src_hbm, dst_hbm, bq_hbm,
             out_sum, out_sq, out_mx, out_mn, out_cnt,
             src_v, dst_v, list_v, acc_s, acc_q, acc_mx, acc_mn,
             cnt_acc, idx_b0, dl_b0, rows_v0, idx_b1, dl_b1, rows_v1,
             sem0, sem1):
    nc_ax = lax.axis_index("c")
    ns_ax = lax.axis_index("s")
    wid = ns_ax * 2 + nc_ax
    base = wid * NPT
    iota = lax.iota(jnp.int32, 16)
    z16 = jnp.zeros((16,), jnp.float32)
    ones16 = jnp.ones((16,), jnp.float32)

    def zcnt(i, carry):
        cnt_acc[pl.ds(i * 16, 16)] = z16
        return carry

    lax.fori_loop(0, NPT // 16, zcnt, 0)

    def chunk(ci, off):
        pltpu.sync_copy(src_hbm.at[pl.ds(ci * ECH, ECH)], src_v)
        pltpu.sync_copy(dst_hbm.at[pl.ds(ci * ECH, ECH)], dst_v)

        def grp(g, off):
            s = src_v[pl.ds(g * 16, 16)]
            d = dst_v[pl.ds(g * 16, 16)]
            dl = d - base
            m = (dl >= 0) & (dl < NPT)
            dlc = jnp.where(m, dl, 0)
            plsc.addupdate_scatter(cnt_acc, [dlc], ones16, mask=m)
            packed = (s << 9) | dlc
            offc = jnp.minimum(off, CAP - 16)
            plsc.store_compressed(list_v.at[pl.ds(offc, 16)], packed, mask=m)
            pc = plsc.all_reduce_population_count(m)[0]
            return off + pc

        return lax.fori_loop(0, NGRP, grp, off)

    off = lax.fori_loop(0, NCHUNK, chunk, jnp.int32(0))
    pltpu.sync_copy(cnt_acc, out_cnt.at[pl.ds(base, NPT)])

    offc = jnp.minimum(off, CAP)
    dummy = jnp.full((16,), NPT, jnp.int32)
    for g in range(2 * GK // 16):
        list_v[pl.ds(offc + g * 16, 16)] = dummy
    npair = (offc + 2 * GK - 1) // (2 * GK)

    for q in range(4):
        def zacc(i, carry):
            for k in range(FQ // 16):
                ks = pl.ds(k * 16, 16)
                acc_s[i, ks] = z16
                acc_q[i, ks] = z16
                acc_mx[i, ks] = jnp.full((16,), -FMAX, jnp.float32)
                acc_mn[i, ks] = jnp.full((16,), FMAX, jnp.float32)
            return carry

        lax.fori_loop(0, NPT + 1, zacc, 0)

        def build(ci, idx_ref, dl_ref):
            eb = ci * GK
            for g in range(GK // 16):
                p = list_v[pl.ds(eb + g * 16, 16)]
                idx_ref[pl.ds(g * 16, 16)] = p >> 9
                dl_ref[pl.ds(g * 16, 16)] = p & 511

        def start(ci, idx_ref, rows_ref, sem_):
            pltpu.async_copy(bq_hbm.at[q].at[idx_ref], rows_ref, sem_)

        def wait(idx_ref, rows_ref, sem_):
            pltpu.make_async_copy(
                bq_hbm.at[q].at[idx_ref], rows_ref, sem_).wait()

        def process(rows_ref, dl_ref):
            def grp16(g, carry):
                base_e = g * 16
                dlg = dl_ref[pl.ds(base_e, 16)]
                for j in range(16):
                    dl = dlg[j]
                    e = base_e + j
                    for k in range(FQ // 16):
                        ks = pl.ds(k * 16, 16)
                        r = rows_ref[e, ks]
                        acc_s[dl, ks] = acc_s[dl, ks] + r
                        acc_q[dl, ks] = acc_q[dl, ks] + r * r
                        acc_mx[dl, ks] = jnp.maximum(acc_mx[dl, ks], r)
                        acc_mn[dl, ks] = jnp.minimum(acc_mn[dl, ks], r)
                return carry

            lax.fori_loop(0, GK // 16, grp16, 0)

        build(0, idx_b0, dl_b0)
        start(0, idx_b0, rows_v0, sem0)

        def pair(pi, carry):
            ci0 = pi * 2
            build(ci0 + 1, idx_b1, dl_b1)
            start(ci0 + 1, idx_b1, rows_v1, sem1)
            wait(idx_b0, rows_v0, sem0)
            process(rows_v0, dl_b0)

            @pl.when(pi + 1 < npair)
            def _():
                build(ci0 + 2, idx_b0, dl_b0)
                start(ci0 + 2, idx_b0, rows_v0, sem0)

            wait(idx_b1, rows_v1, sem1)
            process(rows_v1, dl_b1)
            return carry

        lax.fori_loop(0, npair, pair, 0)

        rs = pl.ds(base, NPT)
        pltpu.sync_copy(acc_s.at[0:NPT], out_sum.at[q].at[rs])
        pltpu.sync_copy(acc_q.at[0:NPT], out_sq.at[q].at[rs])
        pltpu.sync_copy(acc_mx.at[0:NPT], out_mx.at[q].at[rs])
        pltpu.sync_copy(acc_mn.at[0:NPT], out_mn.at[q].at[rs])


def _sc_scatter(src, dst, bq):
    f32 = jnp.float32
    agg_t = jax.ShapeDtypeStruct((4, N_PAD, FQ), f32)
    return pl.kernel(
        _sc_body,
        out_type=(agg_t, agg_t, agg_t, agg_t,
                  jax.ShapeDtypeStruct((N_PAD,), f32)),
        mesh=plsc.VectorSubcoreMesh(core_axis_name="c", subcore_axis_name="s"),
        compiler_params=pltpu.CompilerParams(
            needs_layout_passes=False, use_tc_tiling_on_sc=False),
        scratch_types=[
            pltpu.VMEM((ECH,), jnp.int32),
            pltpu.VMEM((ECH,), jnp.int32),
            pltpu.VMEM((CAP + 2 * GK,), jnp.int32),
            pltpu.VMEM((NPT + 1, FQ), f32),
            pltpu.VMEM((NPT + 1, FQ), f32),
            pltpu.VMEM((NPT + 1, FQ), f32),
            pltpu.VMEM((NPT + 1, FQ), f32),
            pltpu.VMEM((NPT,), f32),
            pltpu.VMEM((GK,), jnp.int32),
            pltpu.VMEM((GK + 16,), jnp.int32),
            pltpu.VMEM((GK, FQ), f32),
            pltpu.VMEM((GK,), jnp.int32),
            pltpu.VMEM((GK + 16,), jnp.int32),
            pltpu.VMEM((GK, FQ), f32),
            pltpu.SemaphoreType.DMA,
            pltpu.SemaphoreType.DMA,
        ],
    )(src, dst, bq)


def _comb_body(x_ref, a_ref, cs_ref, cq_ref, cx_ref, cn_ref, cnt_ref,
               w0_ref, wa_ref, wb_ref, wc_ref, bp_ref, wl_ref, bl_ref, o_ref):
    cnt_raw = cnt_ref[...]
    he = cnt_raw > 0.0
    cnt = jnp.maximum(cnt_raw, 1.0)
    inv = 1.0 / cnt
    mxs, mns, means, stds = [], [], [], []
    for q in range(4):
        a_q = a_ref[:, q * FQ:(q + 1) * FQ]
        gmean = cs_ref[q] * inv
        means.append(jnp.where(he, a_q + gmean, 0.0))
        var = cq_ref[q] * inv - gmean * gmean
        stds.append(jnp.sqrt(jnp.maximum(var, 0.0) + 1e-5))
        mxs.append(jnp.where(he, a_q + cx_ref[q], 0.0))
        mns.append(jnp.where(he, a_q + cn_ref[q], 0.0))
    agg = jnp.concatenate(mxs + mns + means + stds, axis=-1)
    amp = jnp.log(cnt + 1.0)
    att = 1.0 / amp
    dot = functools.partial(jnp.dot, preferred_element_type=jnp.float32)
    h = (dot(x_ref[...], w0_ref[...])
         + dot(agg, wa_ref[...])
         + amp * dot(agg, wb_ref[...])
         + att * dot(agg, wc_ref[...])
         + bp_ref[...])
    o_ref[...] = dot(h, wl_ref[...]) + bl_ref[...]


def _combine(x_pad, a_tab, csum, csq, cmax, cmin, cnt,
             W_post, b_post, W_lin, b_lin, bm=512):
    m = x_pad.shape[0]
    n = W_lin.shape[1]
    blk = lambda r, c: pl.BlockSpec((r, c), lambda i: (i, 0))
    wblk = lambda r, c: pl.BlockSpec((r, c), lambda i: (0, 0))
    qblk = pl.BlockSpec((4, bm, FQ), lambda i: (0, i, 0))
    return pl.pallas_call(
        _comb_body,
        grid=(m // bm,),
        in_specs=[
            blk(bm, F), blk(bm, F), qblk, qblk, qblk, qblk, blk(bm, 1),
            wblk(F, n), wblk(4 * F, n), wblk(4 * F, n), wblk(4 * F, n),
            wblk(1, n), wblk(F, n), wblk(1, n),
        ],
        out_specs=blk(bm, n),
        out_shape=jax.ShapeDtypeStruct((m, n), jnp.float32),
    )(x_pad, a_tab, csum, csq, cmax, cmin, cnt,
      W_post[:F], W_post[F:5 * F], W_post[5 * F:9 * F], W_post[9 * F:],
      b_post.reshape(1, -1), W_lin, b_lin.reshape(1, -1))


def kernel(x, edge_index, W_pre, b_pre, W_post, b_post, W_lin, b_lin):
    n_nodes, f = x.shape
    src = edge_index[0].astype(jnp.int32)
    dst = edge_index[1].astype(jnp.int32)

    x_pad = jnp.pad(x, ((0, N_PAD - n_nodes), (0, 0)))
    w_cat = jnp.concatenate([W_pre[:f], W_pre[f:]], axis=1)
    b_cat = jnp.concatenate([b_pre, jnp.zeros_like(b_pre)])
    ab = _mm(x_pad, w_cat, b_cat)
    a_tab = ab[:, :f]
    b_tab = ab[:, f:]

    bq = b_tab.reshape(N_PAD, 4, FQ).transpose(1, 0, 2)

    csum, csq, cmax, cmin, cnt = _sc_scatter(src, dst, bq)

    out = _combine(x_pad, a_tab, csum, csq, cmax, cmin,
                   cnt.reshape(N_PAD, 1), W_post, b_post, W_lin, b_lin)
    return out[:n_nodes]
